# baseline (speedup 1.0000x reference)
import functools

import jax
import jax.numpy as jnp
from jax import lax
from jax.experimental import pallas as pl
from jax.experimental.pallas import tpu as pltpu
from jax.experimental.pallas import tpu_sc as plsc

_B = 4
_N = 8192
_IN = 64
_OUT = 128
_HID = 64
_STRIDE = 4
_S = _N // _STRIDE
_K = 32
_R2 = 0.2 * 0.2
_NF = 4
_L = 16
_NC, _NS = 2, 16
_NW = _NC * _NS
_WPB = _NW // _B
_CPW = _S // _WPB
_NCHUNK = _N // _L
_UNROLL = 16
_SBLK = 64
_ROWS = _SBLK * _K
_NBLK = _B * _S // _SBLK


def _bf16r(v):
    u = lax.bitcast_convert_type(v, jnp.uint32)
    r = (u + jnp.uint32(0x7FFF) + ((u >> jnp.uint32(16)) & jnp.uint32(1)))
    r = r & jnp.uint32(0xFFFF0000)
    return lax.bitcast_convert_type(r, jnp.float32)


def _bf16bits(v):
    u = lax.bitcast_convert_type(v, jnp.uint32)
    r = (u + jnp.uint32(0x7FFF) + ((u >> jnp.uint32(16)) & jnp.uint32(1)))
    return r & jnp.uint32(0xFFFF0000)


def _sc_ball_query_gather(posx, posy, posz, feat2):
    mesh = plsc.VectorSubcoreMesh(core_axis_name="c", subcore_axis_name="s")
    bsk = _B * _S * _K

    @functools.partial(
        pl.kernel,
        out_type=[
            jax.ShapeDtypeStruct((bsk // 2, 2 * _IN), jnp.float32),
            jax.ShapeDtypeStruct((_NBLK, 2, _ROWS // 2), jnp.float32),
            jax.ShapeDtypeStruct((_NBLK, 2, _ROWS // 2), jnp.float32),
            jax.ShapeDtypeStruct((_NBLK, 2, _ROWS // 2), jnp.float32),
        ],
        mesh=mesh,
        compiler_params=pltpu.CompilerParams(
            needs_layout_passes=False, use_tc_tiling_on_sc=False),
        scratch_types=[
            pltpu.VMEM((_N,), jnp.float32),
            pltpu.VMEM((_N,), jnp.float32),
            pltpu.VMEM((_N,), jnp.float32),
            pltpu.VMEM((_N,), jnp.float32),
            pltpu.VMEM((_N,), jnp.int32),
            pltpu.VMEM((_N,), jnp.float32),
            pltpu.VMEM((_K,), jnp.int32),
            pltpu.VMEM((_K,), jnp.int32),
            pltpu.VMEM((_K,), jnp.int32),
            pltpu.VMEM((_K, _IN), jnp.float32),
            pltpu.VMEM((_K, _IN), jnp.float32),
            pltpu.VMEM((_CPW * _L,), jnp.float32),
            pltpu.VMEM((_CPW * _L,), jnp.float32),
            pltpu.VMEM((_CPW * _L,), jnp.float32),
            pltpu.VMEM((_CPW * _L,), jnp.float32),
            pltpu.VMEM((_CPW * _L,), jnp.float32),
            pltpu.VMEM((_CPW * _L,), jnp.float32),
            pltpu.SemaphoreType.DMA,
            pltpu.SemaphoreType.DMA,
            pltpu.SemaphoreType.DMA,
            pltpu.SemaphoreType.DMA,
        ],
    )
    def k(px_h, py_h, pz_h, f_h, gf_h, rx_h, ry_h, rz_h,
          px, py, pz, nrm, pxy, pzb, buf, gbufA, gbufB, gfvA, gfvB,
          rxe, rxo, rye, ryo, rze, rzo, gsemA, gsemB, osemA, osemB):
        wid = lax.axis_index("s") * _NC + lax.axis_index("c")
        b = wid // _WPB
        ww = wid % _WPB
        wb = b * _S + ww * _CPW

        pltpu.sync_copy(px_h.at[b], px)
        pltpu.sync_copy(py_h.at[b], py)
        pltpu.sync_copy(pz_h.at[b], pz)

        def norm_body(i, carry):
            sl = pl.ds(i * _L, _L)
            x = px[sl]
            y = py[sl]
            z = pz[sl]
            nrm[sl] = x * x + y * y + z * z
            packed = _bf16bits(x) | (_bf16bits(y) >> jnp.uint32(16))
            pxy[sl] = lax.bitcast_convert_type(packed, jnp.int32)
            pzb[sl] = _bf16r(z)
            return carry

        lax.fori_loop(0, _NCHUNK, norm_body, 0)

        lane = lax.iota(jnp.int32, _L)
        zeros = jnp.zeros((_L,), jnp.int32)

        def ballquery(c):
            p = _STRIDE * (ww * _CPW + c)
            pvec = jnp.full((_L,), p, jnp.int32)
            cx = plsc.load_gather(px, [pvec])
            cy = plsc.load_gather(py, [pvec])
            cz = plsc.load_gather(pz, [pvec])
            nc = cx * cx + cy * cy + cz * cz
            cxb = _bf16r(cx)
            cyb = _bf16r(cy)
            czb = _bf16r(cz)

            hi16 = jnp.uint32(0xFFFF0000)

            def cond(c):
                cntv, ch = c
                return jnp.logical_and(jnp.all(cntv < _K),
                                       ch < _NCHUNK // _UNROLL)

            def body(c):
                cntv, ch = c
                base = ch * (_L * _UNROLL)
                masks = []
                for q in range(_UNROLL):
                    sl = pl.ds(base + q * _L, _L)
                    pv = lax.bitcast_convert_type(pxy[sl], jnp.uint32)
                    x = lax.bitcast_convert_type(pv & hi16, jnp.float32)
                    y = lax.bitcast_convert_type(pv << jnp.uint32(16),
                                                 jnp.float32)
                    z = pzb[sl]
                    nv = nrm[sl]
                    dot = x * cxb + y * cyb + z * czb
                    d = (nc + nv) - 2.0 * dot
                    masks.append(d <= _R2)
                for q in range(_UNROLL):
                    m = masks[q]
                    cq = cntv

                    @pl.when(jnp.any(m))
                    def _(m=m, cq=cq, q=q):
                        pref = plsc.cumsum(m.astype(jnp.int32))
                        pos = cq + pref - 1
                        wm = jnp.logical_and(m, pos < _K)
                        idxv = base + q * _L + lane
                        plsc.store_scatter(buf, [jnp.where(wm, pos, 0)],
                                           idxv, mask=wm)

                    cntv = cntv + plsc.all_reduce_population_count(m)
                return (cntv, ch + 1)

            cntv, _ = lax.while_loop(cond, body, (zeros, jnp.int32(0)))

            first = plsc.load_gather(buf, [zeros])
            lane2 = lane * 2
            ie = jnp.where(lane2 < cntv, plsc.load_gather(buf, [lane2]), first)
            io = jnp.where(lane2 + 1 < cntv,
                           plsc.load_gather(buf, [lane2 + 1]), first)
            return ie, io, cx, cy, cz

        def emit_rel(c, ie, io, cx, cy, cz):
            o = c * _L
            rxe[pl.ds(o, _L)] = plsc.load_gather(px, [ie]) - cx
            rxo[pl.ds(o, _L)] = plsc.load_gather(px, [io]) - cx
            rye[pl.ds(o, _L)] = plsc.load_gather(py, [ie]) - cy
            ryo[pl.ds(o, _L)] = plsc.load_gather(py, [io]) - cy
            rze[pl.ds(o, _L)] = plsc.load_gather(pz, [ie]) - cz
            rzo[pl.ds(o, _L)] = plsc.load_gather(pz, [io]) - cz

        def gf_dst(c, half):
            return gf_h.at[pl.ds((wb + c) * _L, _L),
                           pl.ds(half * _IN, _IN)]

        def start_gather(gbuf, gfv, sem, ie, io):
            gbuf[pl.ds(0, _L)] = ie + b * _N
            gbuf[pl.ds(_L, _L)] = io + b * _N
            pltpu.make_async_copy(
                f_h.at[gbuf.at[pl.ds(0, _L)]], gfv.at[pl.ds(0, _L)],
                sem).start()
            pltpu.make_async_copy(
                f_h.at[gbuf.at[pl.ds(_L, _L)]], gfv.at[pl.ds(_L, _L)],
                sem).start()

        def wait_gather(gbuf, gfv, sem):
            pltpu.make_async_copy(
                f_h.at[gbuf.at[pl.ds(0, _L)]], gfv.at[pl.ds(0, _L)],
                sem).wait()
            pltpu.make_async_copy(
                f_h.at[gbuf.at[pl.ds(_L, _L)]], gfv.at[pl.ds(_L, _L)],
                sem).wait()

        def start_out(gfv, sem, c):
            pltpu.make_async_copy(gfv.at[pl.ds(0, _L)],
                                  gf_dst(c, 0), sem).start()
            pltpu.make_async_copy(gfv.at[pl.ds(_L, _L)],
                                  gf_dst(c, 1), sem).start()

        def wait_out(gfv, sem):
            pltpu.make_async_copy(gfv.at[pl.ds(0, _L)],
                                  gf_dst(0, 0), sem).wait()
            pltpu.make_async_copy(gfv.at[pl.ds(_L, _L)],
                                  gf_dst(0, 1), sem).wait()

        def pipe_body(jj, carry):
            cA = 2 * jj
            cB = cA + 1

            iAe, iAo, cax, cay, caz = ballquery(cA)

            @pl.when(jj > 0)
            def _():
                wait_gather(gbufB, gfvB, gsemB)
                start_out(gfvB, osemB, cB - 2)
                wait_out(gfvA, osemA)

            start_gather(gbufA, gfvA, gsemA, iAe, iAo)
            emit_rel(cA, iAe, iAo, cax, cay, caz)

            iBe, iBo, cbx, cby, cbz = ballquery(cB)

            wait_gather(gbufA, gfvA, gsemA)
            start_out(gfvA, osemA, cA)

            @pl.when(jj > 0)
            def _():
                wait_out(gfvB, osemB)

            start_gather(gbufB, gfvB, gsemB, iBe, iBo)
            emit_rel(cB, iBe, iBo, cbx, cby, cbz)
            return carry

        lax.fori_loop(0, _CPW // 2, pipe_body, 0)

        wait_gather(gbufB, gfvB, gsemB)
        start_out(gfvB, osemB, _CPW - 1)
        wait_out(gfvA, osemA)
        wait_out(gfvB, osemB)

        wr0 = wb // _SBLK
        nrow = _CPW // _SBLK
        half = _ROWS // 2
        for r in range(nrow):
            sl = pl.ds(r * half, half)
            pltpu.sync_copy(rxe.at[sl], rx_h.at[wr0 + r, 0])
            pltpu.sync_copy(rxo.at[sl], rx_h.at[wr0 + r, 1])
            pltpu.sync_copy(rye.at[sl], ry_h.at[wr0 + r, 0])
            pltpu.sync_copy(ryo.at[sl], ry_h.at[wr0 + r, 1])
            pltpu.sync_copy(rze.at[sl], rz_h.at[wr0 + r, 0])
            pltpu.sync_copy(rzo.at[sl], rz_h.at[wr0 + r, 1])

    return k(posx, posy, posz, feat2)


def _gelu(x):
    return 0.5 * x * (1.0 + lax.erf(x * 0.7071067811865476))


def _dot_t(a, w):
    return lax.dot_general(a, w, (((0,), (0,)), ((), ())),
                           preferred_element_type=jnp.float32)


def _trig(relT):
    a1 = relT * jnp.float32(jnp.pi)
    s1 = jnp.sin(a1)
    c1 = jnp.cos(a1)
    s2 = 2.0 * s1 * c1
    c2 = 1.0 - 2.0 * s1 * s1
    s4 = 2.0 * s2 * c2
    c4 = 1.0 - 2.0 * s2 * s2
    s8 = 2.0 * s4 * c4
    c8 = 1.0 - 2.0 * s4 * s4
    return (jnp.concatenate([s1, s2, s4, s8], axis=0),
            jnp.concatenate([c1, c2, c4, c8], axis=0))


def _tc_body(gf, rx, ry, rz, feat, pos,
             W1g, W1r, W1s, W1c, b1, g1, be1,
             W2, b2, g2, be2, Wr, br, gr, ber,
             o_pos, o_feat):
    r2x = rx[...][0]
    r2y = ry[...][0]
    r2z = rz[...][0]

    def ln(v, g, be):
        mu = jnp.mean(v, axis=-1, keepdims=True)
        var = jnp.mean((v - mu) ** 2, axis=-1, keepdims=True)
        return (v - mu) * lax.rsqrt(var + 1e-5) * g[...] + be[...]

    def half_stream(gfh, relT):
        sall, call = _trig(relT)
        h = (jnp.dot(gfh, W1g[...], preferred_element_type=jnp.float32)
             + _dot_t(relT, W1r[...]) + _dot_t(sall, W1s[...])
             + _dot_t(call, W1c[...]) + b1[...])
        h = _gelu(ln(h, g1, be1))
        h = jnp.dot(h, W2[...], preferred_element_type=jnp.float32) + b2[...]
        h = ln(h, g2, be2)
        return jnp.max(h.reshape(_SBLK, _K // 2, _OUT), axis=1)

    gfp = gf[...]
    he = half_stream(gfp[:, :_IN],
                     jnp.concatenate([r2x[0:1], r2y[0:1], r2z[0:1]], axis=0))
    ho = half_stream(gfp[:, _IN:],
                     jnp.concatenate([r2x[1:2], r2y[1:2], r2z[1:2]], axis=0))
    h = jnp.maximum(he, ho)

    idb = feat[...][0].reshape(_SBLK, _STRIDE, _IN)[:, 0, :]
    res = jnp.dot(idb, Wr[...], preferred_element_type=jnp.float32) + br[...]
    res = ln(res, gr, ber)

    o_feat[...] = _gelu(h + res)
    o_pos[...] = pos[...][0].reshape(_SBLK, _STRIDE, 3)[:, 0, :]


_SIN_ROWS = tuple(67 + d * 8 + j for j in range(_NF) for d in range(3))
_COS_ROWS = tuple(67 + d * 8 + _NF + j for j in range(_NF) for d in range(3))


def _tc_mlp(gf, rx2, ry2, rz2, feature, position,
            W1, b1, g1, be1, W2, b2, g2, be2, Wr, br, gr, ber):
    bs = _B * _S
    spb = _S // _SBLK
    rep = pl.BlockSpec()
    W1g = W1[3:3 + _IN]
    W1r = W1[0:3]
    W1s = W1[jnp.array(_SIN_ROWS)]
    W1c = W1[jnp.array(_COS_ROWS)]
    out = pl.pallas_call(
        _tc_body,
        grid=(_NBLK,),
        in_specs=[
            pl.BlockSpec((_ROWS // 2, 2 * _IN), lambda i: (i, 0)),
            pl.BlockSpec((1, 2, _ROWS // 2), lambda i: (i, 0, 0)),
            pl.BlockSpec((1, 2, _ROWS // 2), lambda i: (i, 0, 0)),
            pl.BlockSpec((1, 2, _ROWS // 2), lambda i: (i, 0, 0)),
            pl.BlockSpec((1, _SBLK * _STRIDE, _IN),
                         lambda i: (i // spb, i % spb, 0)),
            pl.BlockSpec((1, _SBLK * _STRIDE, 3),
                         lambda i: (i // spb, i % spb, 0)),
            rep, rep, rep, rep, rep, rep, rep,
            rep, rep, rep, rep, rep, rep, rep, rep,
        ],
        out_specs=[
            pl.BlockSpec((_SBLK, 3), lambda i: (i, 0)),
            pl.BlockSpec((_SBLK, _OUT), lambda i: (i, 0)),
        ],
        out_shape=[
            jax.ShapeDtypeStruct((bs, 3), jnp.float32),
            jax.ShapeDtypeStruct((bs, _OUT), jnp.float32),
        ],
    )(gf, rx2, ry2, rz2, feature, position,
      W1g, W1r, W1s, W1c,
      b1.reshape(1, -1), g1.reshape(1, -1), be1.reshape(1, -1),
      W2, b2.reshape(1, -1), g2.reshape(1, -1), be2.reshape(1, -1),
      Wr, br.reshape(1, -1), gr.reshape(1, -1), ber.reshape(1, -1))
    return out


def kernel(position, feature, W1, b1, g1, be1, W2, b2, g2, be2,
           Wr, br, gr, ber):
    posx = position[..., 0]
    posy = position[..., 1]
    posz = position[..., 2]
    feat2 = feature.reshape(_B * _N, _IN)

    gf, rxf, ryf, rzf = _sc_ball_query_gather(posx, posy, posz, feat2)

    new_pos, new_feat = _tc_mlp(gf, rxf, ryf, rzf, feature, position,
                                W1, b1, g1, be1, W2, b2, g2, be2,
                                Wr, br, gr, ber)
    return new_pos.reshape(_B, _S, 3), new_feat.reshape(_B, _S, _OUT)


# speedup vs baseline: 1.2255x; 1.2255x over previous
import functools

import jax
import jax.numpy as jnp
from jax import lax
from jax.experimental import pallas as pl
from jax.experimental.pallas import tpu as pltpu
from jax.experimental.pallas import tpu_sc as plsc

_B = 4
_N = 8192
_IN = 64
_OUT = 128
_HID = 64
_STRIDE = 4
_S = _N // _STRIDE
_K = 32
_R2 = 0.2 * 0.2
_NF = 4
_L = 16
_NC, _NS = 2, 16
_NW = _NC * _NS
_WPB = _NW // _B
_CPW = _S // _WPB
_NCHUNK = _N // _L
_UNROLL = 8
_SBLK = 64
_ROWS = _SBLK * _K
_NBLK = _B * _S // _SBLK
_CPH = _CPW // 2


def _bf16r(v):
    u = lax.bitcast_convert_type(v, jnp.uint32)
    r = (u + jnp.uint32(0x7FFF) + ((u >> jnp.uint32(16)) & jnp.uint32(1)))
    r = r & jnp.uint32(0xFFFF0000)
    return lax.bitcast_convert_type(r, jnp.float32)


def _bf16bits(v):
    u = lax.bitcast_convert_type(v, jnp.uint32)
    r = (u + jnp.uint32(0x7FFF) + ((u >> jnp.uint32(16)) & jnp.uint32(1)))
    return r & jnp.uint32(0xFFFF0000)


def _sc_ball_query_gather(posx, posy, posz, feat2, phase):
    mesh = plsc.VectorSubcoreMesh(core_axis_name="c", subcore_axis_name="s")
    bsk = _B * _S * _K

    @functools.partial(
        pl.kernel,
        out_type=[
            jax.ShapeDtypeStruct((bsk // 4, 2 * _IN), jnp.float32),
            jax.ShapeDtypeStruct((_NBLK // 2, 2, _ROWS // 2), jnp.float32),
            jax.ShapeDtypeStruct((_NBLK // 2, 2, _ROWS // 2), jnp.float32),
            jax.ShapeDtypeStruct((_NBLK // 2, 2, _ROWS // 2), jnp.float32),
        ],
        mesh=mesh,
        compiler_params=pltpu.CompilerParams(
            needs_layout_passes=False, use_tc_tiling_on_sc=False),
        scratch_types=[
            pltpu.VMEM((_N,), jnp.float32),
            pltpu.VMEM((_N,), jnp.float32),
            pltpu.VMEM((_N,), jnp.float32),
            pltpu.VMEM((_N,), jnp.float32),
            pltpu.VMEM((_N,), jnp.int32),
            pltpu.VMEM((_N,), jnp.float32),
            pltpu.VMEM((_K,), jnp.int32),
            pltpu.VMEM((_K,), jnp.int32),
            pltpu.VMEM((_K,), jnp.int32),
            pltpu.VMEM((_K, _IN), jnp.float32),
            pltpu.VMEM((_K, _IN), jnp.float32),
            pltpu.VMEM((_CPH * _L,), jnp.float32),
            pltpu.VMEM((_CPH * _L,), jnp.float32),
            pltpu.VMEM((_CPH * _L,), jnp.float32),
            pltpu.VMEM((_CPH * _L,), jnp.float32),
            pltpu.VMEM((_CPH * _L,), jnp.float32),
            pltpu.VMEM((_CPH * _L,), jnp.float32),
            pltpu.SemaphoreType.DMA,
            pltpu.SemaphoreType.DMA,
            pltpu.SemaphoreType.DMA,
            pltpu.SemaphoreType.DMA,
        ],
    )
    def k(px_h, py_h, pz_h, f_h, gf_h, rx_h, ry_h, rz_h,
          px, py, pz, nrm, pxy, pzb, buf, gbufA, gbufB, gfvA, gfvB,
          rxe, rxo, rye, ryo, rze, rzo, gsemA, gsemB, osemA, osemB):
        wid = lax.axis_index("s") * _NC + lax.axis_index("c")
        b = wid // _WPB
        ww = wid % _WPB
        pb = (b * _WPB + ww) * _CPH

        pltpu.sync_copy(px_h.at[b], px)
        pltpu.sync_copy(py_h.at[b], py)
        pltpu.sync_copy(pz_h.at[b], pz)

        def norm_body(i, carry):
            sl = pl.ds(i * _L, _L)
            x = px[sl]
            y = py[sl]
            z = pz[sl]
            nrm[sl] = x * x + y * y + z * z
            packed = _bf16bits(x) | (_bf16bits(y) >> jnp.uint32(16))
            pxy[sl] = lax.bitcast_convert_type(packed, jnp.int32)
            pzb[sl] = _bf16r(z)
            return carry

        lax.fori_loop(0, _NCHUNK, norm_body, 0)

        lane = lax.iota(jnp.int32, _L)
        zeros = jnp.zeros((_L,), jnp.int32)

        def ballquery(c):
            p = _STRIDE * (ww * _CPW + phase * _CPH + c)
            pvec = jnp.full((_L,), p, jnp.int32)
            cx = plsc.load_gather(px, [pvec])
            cy = plsc.load_gather(py, [pvec])
            cz = plsc.load_gather(pz, [pvec])
            nc = cx * cx + cy * cy + cz * cz
            cxb = _bf16r(cx)
            cyb = _bf16r(cy)
            czb = _bf16r(cz)

            hi16 = jnp.uint32(0xFFFF0000)

            def cond(c):
                cntv, ch = c
                return jnp.logical_and(jnp.all(cntv < _K),
                                       ch < _NCHUNK // _UNROLL)

            def body(c):
                cntv, ch = c
                base = ch * (_L * _UNROLL)
                masks = []
                for q in range(_UNROLL):
                    sl = pl.ds(base + q * _L, _L)
                    pv = lax.bitcast_convert_type(pxy[sl], jnp.uint32)
                    x = lax.bitcast_convert_type(pv & hi16, jnp.float32)
                    y = lax.bitcast_convert_type(pv << jnp.uint32(16),
                                                 jnp.float32)
                    z = pzb[sl]
                    nv = nrm[sl]
                    dot = x * cxb + y * cyb + z * czb
                    d = (nc + nv) - 2.0 * dot
                    masks.append(d <= _R2)
                for q in range(_UNROLL):
                    m = masks[q]
                    cq = cntv

                    @pl.when(jnp.any(m))
                    def _(m=m, cq=cq, q=q):
                        pref = plsc.cumsum(m.astype(jnp.int32))
                        pos = cq + pref - 1
                        wm = jnp.logical_and(m, pos < _K)
                        idxv = base + q * _L + lane
                        plsc.store_scatter(buf, [jnp.where(wm, pos, 0)],
                                           idxv, mask=wm)

                    cntv = cntv + plsc.all_reduce_population_count(m)
                return (cntv, ch + 1)

            cntv, _ = lax.while_loop(cond, body, (zeros, jnp.int32(0)))

            first = plsc.load_gather(buf, [zeros])
            lane2 = lane * 2
            ie = jnp.where(lane2 < cntv, plsc.load_gather(buf, [lane2]), first)
            io = jnp.where(lane2 + 1 < cntv,
                           plsc.load_gather(buf, [lane2 + 1]), first)
            return ie, io, cx, cy, cz

        def emit_rel(c, ie, io, cx, cy, cz):
            o = c * _L
            rxe[pl.ds(o, _L)] = plsc.load_gather(px, [ie]) - cx
            rxo[pl.ds(o, _L)] = plsc.load_gather(px, [io]) - cx
            rye[pl.ds(o, _L)] = plsc.load_gather(py, [ie]) - cy
            ryo[pl.ds(o, _L)] = plsc.load_gather(py, [io]) - cy
            rze[pl.ds(o, _L)] = plsc.load_gather(pz, [ie]) - cz
            rzo[pl.ds(o, _L)] = plsc.load_gather(pz, [io]) - cz

        def gf_dst(c, half):
            return gf_h.at[pl.ds((pb + c) * _L, _L),
                           pl.ds(half * _IN, _IN)]

        def start_gather(gbuf, gfv, sem, ie, io):
            gbuf[pl.ds(0, _L)] = ie + b * _N
            gbuf[pl.ds(_L, _L)] = io + b * _N
            pltpu.make_async_copy(
                f_h.at[gbuf.at[pl.ds(0, _L)]], gfv.at[pl.ds(0, _L)],
                sem).start()
            pltpu.make_async_copy(
                f_h.at[gbuf.at[pl.ds(_L, _L)]], gfv.at[pl.ds(_L, _L)],
                sem).start()

        def wait_gather(gbuf, gfv, sem):
            pltpu.make_async_copy(
                f_h.at[gbuf.at[pl.ds(0, _L)]], gfv.at[pl.ds(0, _L)],
                sem).wait()
            pltpu.make_async_copy(
                f_h.at[gbuf.at[pl.ds(_L, _L)]], gfv.at[pl.ds(_L, _L)],
                sem).wait()

        def start_out(gfv, sem, c):
            pltpu.make_async_copy(gfv.at[pl.ds(0, _L)],
                                  gf_dst(c, 0), sem).start()
            pltpu.make_async_copy(gfv.at[pl.ds(_L, _L)],
                                  gf_dst(c, 1), sem).start()

        def wait_out(gfv, sem):
            pltpu.make_async_copy(gfv.at[pl.ds(0, _L)],
                                  gf_dst(0, 0), sem).wait()
            pltpu.make_async_copy(gfv.at[pl.ds(_L, _L)],
                                  gf_dst(0, 1), sem).wait()

        def pipe_body(jj, carry):
            cA = 2 * jj
            cB = cA + 1

            iAe, iAo, cax, cay, caz = ballquery(cA)

            @pl.when(jj > 0)
            def _():
                wait_gather(gbufB, gfvB, gsemB)
                start_out(gfvB, osemB, cB - 2)
                wait_out(gfvA, osemA)

            start_gather(gbufA, gfvA, gsemA, iAe, iAo)
            emit_rel(cA, iAe, iAo, cax, cay, caz)

            iBe, iBo, cbx, cby, cbz = ballquery(cB)

            wait_gather(gbufA, gfvA, gsemA)
            start_out(gfvA, osemA, cA)

            @pl.when(jj > 0)
            def _():
                wait_out(gfvB, osemB)

            start_gather(gbufB, gfvB, gsemB, iBe, iBo)
            emit_rel(cB, iBe, iBo, cbx, cby, cbz)
            return carry

        lax.fori_loop(0, _CPH // 2, pipe_body, 0)

        wait_gather(gbufB, gfvB, gsemB)
        start_out(gfvB, osemB, _CPH - 1)
        wait_out(gfvA, osemA)
        wait_out(gfvB, osemB)

        wr0 = pb // _SBLK
        nrow = _CPH // _SBLK
        half = _ROWS // 2
        for r in range(nrow):
            sl = pl.ds(r * half, half)
            pltpu.sync_copy(rxe.at[sl], rx_h.at[wr0 + r, 0])
            pltpu.sync_copy(rxo.at[sl], rx_h.at[wr0 + r, 1])
            pltpu.sync_copy(rye.at[sl], ry_h.at[wr0 + r, 0])
            pltpu.sync_copy(ryo.at[sl], ry_h.at[wr0 + r, 1])
            pltpu.sync_copy(rze.at[sl], rz_h.at[wr0 + r, 0])
            pltpu.sync_copy(rzo.at[sl], rz_h.at[wr0 + r, 1])

    return k(posx, posy, posz, feat2)


def _gelu(x):
    return 0.5 * x * (1.0 + lax.erf(x * 0.7071067811865476))


def _dot_t(a, w):
    return lax.dot_general(a, w, (((0,), (0,)), ((), ())),
                           preferred_element_type=jnp.float32)


def _trig(relT):
    a1 = relT * jnp.float32(jnp.pi)
    s1 = jnp.sin(a1)
    c1 = jnp.cos(a1)
    s2 = 2.0 * s1 * c1
    c2 = 1.0 - 2.0 * s1 * s1
    s4 = 2.0 * s2 * c2
    c4 = 1.0 - 2.0 * s2 * s2
    s8 = 2.0 * s4 * c4
    c8 = 1.0 - 2.0 * s4 * s4
    return (jnp.concatenate([s1, s2, s4, s8], axis=0),
            jnp.concatenate([c1, c2, c4, c8], axis=0))


def _tc_body(gf, rx, ry, rz, feat, pos,
             W1g, W1r, W1s, W1c, b1, g1, be1,
             W2, b2, g2, be2, Wr, br, gr, ber,
             o_pos, o_feat):
    r2x = rx[...][0]
    r2y = ry[...][0]
    r2z = rz[...][0]

    def ln(v, g, be):
        mu = jnp.mean(v, axis=-1, keepdims=True)
        var = jnp.mean((v - mu) ** 2, axis=-1, keepdims=True)
        return (v - mu) * lax.rsqrt(var + 1e-5) * g[...] + be[...]

    def half_stream(gfh, relT):
        sall, call = _trig(relT)
        h = (jnp.dot(gfh, W1g[...], preferred_element_type=jnp.float32)
             + _dot_t(relT, W1r[...]) + _dot_t(sall, W1s[...])
             + _dot_t(call, W1c[...]) + b1[...])
        h = _gelu(ln(h, g1, be1))
        h = jnp.dot(h, W2[...], preferred_element_type=jnp.float32) + b2[...]
        h = ln(h, g2, be2)
        return jnp.max(h.reshape(_SBLK, _K // 2, _OUT), axis=1)

    gfp = gf[...]
    he = half_stream(gfp[:, :_IN],
                     jnp.concatenate([r2x[0:1], r2y[0:1], r2z[0:1]], axis=0))
    ho = half_stream(gfp[:, _IN:],
                     jnp.concatenate([r2x[1:2], r2y[1:2], r2z[1:2]], axis=0))
    h = jnp.maximum(he, ho)

    idb = feat[...][0].reshape(_SBLK, _STRIDE, _IN)[:, 0, :]
    res = jnp.dot(idb, Wr[...], preferred_element_type=jnp.float32) + br[...]
    res = ln(res, gr, ber)

    o_feat[...] = _gelu(h + res)
    o_pos[...] = pos[...][0].reshape(_SBLK, _STRIDE, 3)[:, 0, :]


_SIN_ROWS = tuple(67 + d * 8 + j for j in range(_NF) for d in range(3))
_COS_ROWS = tuple(67 + d * 8 + _NF + j for j in range(_NF) for d in range(3))


def _tc_mlp(gf, rx2, ry2, rz2, feature, position,
            W1, b1, g1, be1, W2, b2, g2, be2, Wr, br, gr, ber, phase):
    bs = _B * _S
    spb = _S // _SBLK
    bpw = _CPH // _SBLK
    wpp = _CPW // _SBLK
    rep = pl.BlockSpec()
    W1g = W1[3:3 + _IN]
    W1r = W1[0:3]
    W1s = W1[jnp.array(_SIN_ROWS)]
    W1c = W1[jnp.array(_COS_ROWS)]

    def gmap(i):
        return (i // bpw) * wpp + phase * bpw + i % bpw

    out = pl.pallas_call(
        _tc_body,
        grid=(_NBLK // 2,),
        in_specs=[
            pl.BlockSpec((_ROWS // 2, 2 * _IN), lambda i: (i, 0)),
            pl.BlockSpec((1, 2, _ROWS // 2), lambda i: (i, 0, 0)),
            pl.BlockSpec((1, 2, _ROWS // 2), lambda i: (i, 0, 0)),
            pl.BlockSpec((1, 2, _ROWS // 2), lambda i: (i, 0, 0)),
            pl.BlockSpec((1, _SBLK * _STRIDE, _IN),
                         lambda i: (gmap(i) // spb, gmap(i) % spb, 0)),
            pl.BlockSpec((1, _SBLK * _STRIDE, 3),
                         lambda i: (gmap(i) // spb, gmap(i) % spb, 0)),
            rep, rep, rep, rep, rep, rep, rep,
            rep, rep, rep, rep, rep, rep, rep, rep,
        ],
        out_specs=[
            pl.BlockSpec((_SBLK, 3), lambda i: (i, 0)),
            pl.BlockSpec((_SBLK, _OUT), lambda i: (i, 0)),
        ],
        out_shape=[
            jax.ShapeDtypeStruct((bs // 2, 3), jnp.float32),
            jax.ShapeDtypeStruct((bs // 2, _OUT), jnp.float32),
        ],
    )(gf, rx2, ry2, rz2, feature, position,
      W1g, W1r, W1s, W1c,
      b1.reshape(1, -1), g1.reshape(1, -1), be1.reshape(1, -1),
      W2, b2.reshape(1, -1), g2.reshape(1, -1), be2.reshape(1, -1),
      Wr, br.reshape(1, -1), gr.reshape(1, -1), ber.reshape(1, -1))
    return out


def kernel(position, feature, W1, b1, g1, be1, W2, b2, g2, be2,
           Wr, br, gr, ber):
    posx = position[..., 0]
    posy = position[..., 1]
    posz = position[..., 2]
    feat2 = feature.reshape(_B * _N, _IN)

    halves = []
    for phase in (0, 1):
        gf, rxf, ryf, rzf = _sc_ball_query_gather(
            posx, posy, posz, feat2, phase)
        halves.append(_tc_mlp(gf, rxf, ryf, rzf, feature, position,
                              W1, b1, g1, be1, W2, b2, g2, be2,
                              Wr, br, gr, ber, phase))
    (np0, nf0), (np1, nf1) = halves
    new_pos = jnp.stack(
        [np0.reshape(_NW, _CPH, 3), np1.reshape(_NW, _CPH, 3)], axis=1)
    new_feat = jnp.stack(
        [nf0.reshape(_NW, _CPH, _OUT), nf1.reshape(_NW, _CPH, _OUT)], axis=1)
    return (new_pos.reshape(_B, _S, 3), new_feat.reshape(_B, _S, _OUT))
